# baseline (speedup 1.0000x reference)
import functools

import jax
import jax.numpy as jnp
from jax import lax
from jax.experimental import pallas as pl
from jax.experimental.pallas import tpu as pltpu
from jax.experimental.pallas import tpu_sc as plsc

E = 8
TOPK = 2
D = 1024
H = 1024
N = 4096
EPS = 1e-5

BM = 256
P = TOPK * N + E * BM
NBLK = P // BM
NA = TOPK * N
CS = 512


def _router_body(x_ref, wgt_ref, logits_ref, wbcf_ref, dest_ref, poff_ref,
                 eid_ref):
    x = x_ref[...]
    logits = jnp.dot(x, wgt_ref[...], preferred_element_type=jnp.float32)
    logits_ref[...] = logits
    m = jnp.max(logits, axis=1, keepdims=True)
    p = jnp.exp(logits - m)
    p = p / jnp.sum(p, axis=1, keepdims=True)
    idx = lax.broadcasted_iota(jnp.int32, (N, E), 1)
    v0 = jnp.max(p, axis=1, keepdims=True)
    i0 = jnp.min(jnp.where(p == v0, idx, E), axis=1, keepdims=True)
    p1 = jnp.where(idx == i0, -jnp.inf, p)
    v1 = jnp.max(p1, axis=1, keepdims=True)
    i1 = jnp.min(jnp.where(p1 == v1, idx, E), axis=1, keepdims=True)
    s = v0 + v1
    w0 = v0 / s
    w1 = v1 / s
    oh0 = (idx == i0).astype(jnp.float32)
    oh1 = (idx == i1).astype(jnp.float32)
    mask = oh0 + oh1

    r = lax.broadcasted_iota(jnp.int32, (CS, CS), 0)
    c = lax.broadcasted_iota(jnp.int32, (CS, CS), 1)
    tri = (c < r).astype(jnp.float32)
    run = jnp.zeros((1, E), jnp.float32)
    chunks = []
    for ci in range(N // CS):
        mc = mask[ci * CS:(ci + 1) * CS, :]
        chunks.append(jnp.dot(tri, mc, preferred_element_type=jnp.float32)
                      + run)
        run = run + jnp.sum(mc, axis=0, keepdims=True)
    cum = jnp.concatenate(chunks, axis=0)

    pcnt = jnp.ceil(run / BM) * BM
    r8 = lax.broadcasted_iota(jnp.int32, (E, E), 0)
    c8 = lax.broadcasted_iota(jnp.int32, (E, E), 1)
    m8 = (r8 < c8).astype(jnp.float32)
    poff = jnp.dot(pcnt, m8, preferred_element_type=jnp.float32)
    poff_ref[...] = poff.astype(jnp.int32)

    pos = cum + poff
    d0 = jnp.sum(oh0 * pos, axis=1, keepdims=True)
    d1 = jnp.sum(oh1 * pos, axis=1, keepdims=True)
    lane = lax.broadcasted_iota(jnp.int32, (N, E), 1)
    dest_ref[...] = (jnp.where(lane == 0, d0, 0.0)
                     + jnp.where(lane == 1, d1, 0.0)).astype(jnp.int32)
    wpair = jnp.stack([jnp.broadcast_to(w0, (N, _WL)),
                       jnp.broadcast_to(w1, (N, _WL))], axis=1)
    wbcf_ref[...] = wpair.reshape(NA, _WL)

    be = lax.broadcasted_iota(jnp.int32, (_WL, E), 0) * BM
    le = lax.broadcasted_iota(jnp.int32, (_WL, E), 1)
    hit = (be.astype(jnp.float32) >= poff) & (le >= 1)
    eid_ref[...] = jnp.sum(hit.astype(jnp.int32), axis=1, keepdims=True)


def _router(x, Wg):
    return pl.pallas_call(
        _router_body,
        out_shape=[
            jax.ShapeDtypeStruct((N, E), jnp.float32),
            jax.ShapeDtypeStruct((NA, _WL), jnp.float32),
            jax.ShapeDtypeStruct((N, E), jnp.int32),
            jax.ShapeDtypeStruct((1, E), jnp.int32),
            jax.ShapeDtypeStruct((_WL, 1), jnp.int32),
        ],
    )(x, Wg.T)


_NC, _NS, _L = 2, 16, 16
_WL = 128
_NW = _NC * _NS
_APW = NA // _NW
_CH = 32


def _dispatch_body(x_hbm, destf_hbm, wbcf_hbm, xg_hbm, wbuf_hbm,
                   dest_v0, dest_v1, tok_v0, tok_v1, rows_v0, rows_v1,
                   wrows_v0, wrows_v1, sem_g0, sem_g1, sem_s0, sem_s1,
                   sem_w0, sem_w1):
    dest_v = (dest_v0, dest_v1)
    tok_v = (tok_v0, tok_v1)
    rows_v = (rows_v0, rows_v1)
    wrows_v = (wrows_v0, wrows_v1)
    sem_g = (sem_g0, sem_g1)
    sem_s = (sem_s0, sem_s1)
    sem_w = (sem_w0, sem_w1)
    wid = lax.axis_index("s") * _NC + lax.axis_index("c")
    base0 = wid * _APW
    nch = _APW // _CH
    gcp = [None] * nch
    scp = [None] * nch
    wcp = [None] * nch

    def fire(ch):
        b = ch % 2
        base = base0 + ch * _CH
        pltpu.sync_copy(destf_hbm.at[pl.ds(base, _CH)], dest_v[b])
        pltpu.sync_copy(wbcf_hbm.at[pl.ds(base, _CH)], wrows_v[b])
        for k in range(_CH // _L):
            tok_v[b][pl.ds(k * _L, _L)] = lax.shift_right_logical(
                base + k * _L + lax.iota(jnp.int32, _L), 1)
        gcp[ch] = pltpu.async_copy(x_hbm.at[tok_v[b]], rows_v[b], sem_g[b])

    fire(0)
    for ch in range(nch):
        if ch + 1 < nch:
            if ch - 1 >= 0:
                scp[ch - 1].wait()
                wcp[ch - 1].wait()
            fire(ch + 1)
        b = ch % 2
        gcp[ch].wait()
        scp[ch] = pltpu.async_copy(rows_v[b], xg_hbm.at[dest_v[b]], sem_s[b])
        wcp[ch] = pltpu.async_copy(wrows_v[b], wbuf_hbm.at[dest_v[b]],
                                   sem_w[b])
    if nch >= 2:
        scp[nch - 2].wait()
        wcp[nch - 2].wait()
    scp[nch - 1].wait()
    wcp[nch - 1].wait()


def _dispatch(x, destf, wbcf):
    mesh = plsc.VectorSubcoreMesh(core_axis_name="c", subcore_axis_name="s",
                                  num_cores=_NC, num_subcores=_NS)
    f = functools.partial(
        pl.kernel,
        out_type=[jax.ShapeDtypeStruct((P, D), jnp.float32),
                  jax.ShapeDtypeStruct((P, _WL), jnp.float32)],
        mesh=mesh,
        scratch_types=[
            pltpu.VMEM((_CH,), jnp.int32),
            pltpu.VMEM((_CH,), jnp.int32),
            pltpu.VMEM((_CH,), jnp.int32),
            pltpu.VMEM((_CH,), jnp.int32),
            pltpu.VMEM((_CH, D), jnp.float32),
            pltpu.VMEM((_CH, D), jnp.float32),
            pltpu.VMEM((_CH, _WL), jnp.float32),
            pltpu.VMEM((_CH, _WL), jnp.float32),
            pltpu.SemaphoreType.DMA,
            pltpu.SemaphoreType.DMA,
            pltpu.SemaphoreType.DMA,
            pltpu.SemaphoreType.DMA,
            pltpu.SemaphoreType.DMA,
            pltpu.SemaphoreType.DMA,
        ],
    )(_dispatch_body)
    return f(x, destf, wbcf)


def _ffn_body(eid_ref, xg_ref, w_ref, w1_ref, b1_ref, gamma_ref, beta_ref,
              rmean_ref, rvar_ref, w2_ref, b2_ref, obuf_ref):
    xb = xg_ref[...].astype(jnp.bfloat16)
    h = lax.dot_general(xb, w1_ref[0], (((1,), (1,)), ((), ())),
                        preferred_element_type=jnp.float32)
    scale = gamma_ref[0] * lax.rsqrt(rvar_ref[0] + EPS)
    h = (h + b1_ref[0] - rmean_ref[0]) * scale + beta_ref[0]
    h = jnp.maximum(h, 0.0).astype(jnp.bfloat16)
    o = lax.dot_general(h, w2_ref[0], (((1,), (1,)), ((), ())),
                        preferred_element_type=jnp.float32) + b2_ref[0]
    obuf_ref[...] = o * w_ref[:, 0:1]


def _ffn(eid, xg, wbuf, W1, b1, gamma, beta, rmean, rvar, W2, b2):
    grid_spec = pltpu.PrefetchScalarGridSpec(
        num_scalar_prefetch=1,
        grid=(NBLK,),
        in_specs=[
            pl.BlockSpec((BM, D), lambda b, eid: (b, 0)),
            pl.BlockSpec((BM, _WL), lambda b, eid: (b, 0)),
            pl.BlockSpec((1, H, D), lambda b, eid: (eid[b], 0, 0)),
            pl.BlockSpec((1, 1, H), lambda b, eid: (eid[b], 0, 0)),
            pl.BlockSpec((1, 1, H), lambda b, eid: (eid[b], 0, 0)),
            pl.BlockSpec((1, 1, H), lambda b, eid: (eid[b], 0, 0)),
            pl.BlockSpec((1, 1, H), lambda b, eid: (eid[b], 0, 0)),
            pl.BlockSpec((1, 1, H), lambda b, eid: (eid[b], 0, 0)),
            pl.BlockSpec((1, D, H), lambda b, eid: (eid[b], 0, 0)),
            pl.BlockSpec((1, 1, D), lambda b, eid: (eid[b], 0, 0)),
        ],
        out_specs=pl.BlockSpec((BM, D), lambda b, eid: (b, 0)),
    )
    return pl.pallas_call(
        _ffn_body,
        grid_spec=grid_spec,
        out_shape=jax.ShapeDtypeStruct((P, D), jnp.float32),
    )(eid, xg, wbuf, W1.astype(jnp.bfloat16), b1[:, None], gamma[:, None],
      beta[:, None], rmean[:, None], rvar[:, None], W2.astype(jnp.bfloat16),
      b2[:, None])


_TPW = N // _NW
_CT = 16


def _combine_body(x_hbm, obuf_hbm, d0_hbm, d1_hbm, fin_hbm,
                  d0_v0, d0_v1, d1_v0, d1_v1, x_v0, x_v1, r0_v0, r0_v1,
                  r1_v0, r1_v1, sem_x0, sem_x1, sem_00, sem_01, sem_10,
                  sem_11, sem_f0, sem_f1):
    d0_v = (d0_v0, d0_v1)
    d1_v = (d1_v0, d1_v1)
    x_v = (x_v0, x_v1)
    r0_v = (r0_v0, r0_v1)
    r1_v = (r1_v0, r1_v1)
    sem_x = (sem_x0, sem_x1)
    sem_0 = (sem_00, sem_01)
    sem_1 = (sem_10, sem_11)
    sem_f = (sem_f0, sem_f1)
    wid = lax.axis_index("s") * _NC + lax.axis_index("c")
    tok0 = wid * _TPW
    nch = _TPW // _CT
    gx = [None] * nch
    g0 = [None] * nch
    g1 = [None] * nch
    wb = [None] * nch

    def fire(ch):
        b = ch % 2
        base = tok0 + ch * _CT
        pltpu.sync_copy(d0_hbm.at[pl.ds(base, _CT)], d0_v[b])
        pltpu.sync_copy(d1_hbm.at[pl.ds(base, _CT)], d1_v[b])
        gx[ch] = pltpu.async_copy(x_hbm.at[pl.ds(base, _CT)], x_v[b],
                                  sem_x[b])
        g0[ch] = pltpu.async_copy(obuf_hbm.at[d0_v[b]], r0_v[b], sem_0[b])
        g1[ch] = pltpu.async_copy(obuf_hbm.at[d1_v[b]], r1_v[b], sem_1[b])

    fire(0)
    for ch in range(nch):
        if ch + 1 < nch:
            if ch - 1 >= 0:
                wb[ch - 1].wait()
            fire(ch + 1)
        b = ch % 2
        gx[ch].wait()
        g0[ch].wait()
        g1[ch].wait()

        def body(j, _):
            for k in range(D // _L):
                sl = pl.ds(k * _L, _L)
                x_v[b][j, sl] = x_v[b][j, sl] + r0_v[b][j, sl] + r1_v[b][j, sl]
            return 0

        lax.fori_loop(0, _CT, body, 0)
        wb[ch] = pltpu.async_copy(x_v[b], fin_hbm.at[pl.ds(tok0 + ch * _CT,
                                                           _CT)], sem_f[b])
    if nch >= 2:
        wb[nch - 2].wait()
    wb[nch - 1].wait()


def _combine(x, obuf, d0, d1):
    mesh = plsc.VectorSubcoreMesh(core_axis_name="c", subcore_axis_name="s",
                                  num_cores=_NC, num_subcores=_NS)
    f = functools.partial(
        pl.kernel,
        out_type=jax.ShapeDtypeStruct((N, D), jnp.float32),
        mesh=mesh,
        scratch_types=[
            pltpu.VMEM((_CT,), jnp.int32),
            pltpu.VMEM((_CT,), jnp.int32),
            pltpu.VMEM((_CT,), jnp.int32),
            pltpu.VMEM((_CT,), jnp.int32),
            pltpu.VMEM((_CT, D), jnp.float32),
            pltpu.VMEM((_CT, D), jnp.float32),
            pltpu.VMEM((_CT, D), jnp.float32),
            pltpu.VMEM((_CT, D), jnp.float32),
            pltpu.VMEM((_CT, D), jnp.float32),
            pltpu.VMEM((_CT, D), jnp.float32),
            pltpu.SemaphoreType.DMA,
            pltpu.SemaphoreType.DMA,
            pltpu.SemaphoreType.DMA,
            pltpu.SemaphoreType.DMA,
            pltpu.SemaphoreType.DMA,
            pltpu.SemaphoreType.DMA,
            pltpu.SemaphoreType.DMA,
            pltpu.SemaphoreType.DMA,
        ],
    )(_combine_body)
    return f(x, obuf, d0, d1)


def kernel(hidden_states, Wg, W1, b1, gamma, beta, rmean, rvar, W2, b2):
    x = hidden_states
    logits, wbcf, dest, poff, eidc = _router(x, Wg)

    destf = dest[:, :TOPK].reshape(-1)
    d0 = dest[:, 0]
    d1 = dest[:, 1]
    eid = eidc[:NBLK, 0]

    xg, wbuf = _dispatch(x, destf, wbcf)
    obuf = _ffn(eid, xg, wbuf, W1, b1, gamma, beta, rmean, rvar, W2, b2)
    final = _combine(x, obuf, d0, d1)
    return final, logits


# speedup vs baseline: 1.1087x; 1.1087x over previous
import functools

import jax
import jax.numpy as jnp
from jax import lax
from jax.experimental import pallas as pl
from jax.experimental.pallas import tpu as pltpu
from jax.experimental.pallas import tpu_sc as plsc

E = 8
TOPK = 2
D = 1024
H = 1024
N = 4096
EPS = 1e-5

BM = 256
P = TOPK * N + E * BM
NBLK = P // BM
NA = TOPK * N
CS = 512


def _router_body(x_ref, wgt_ref, logits_ref, wbcf_ref, dest_ref, poff_ref,
                 eid_ref):
    x = x_ref[...]
    logits = jnp.dot(x, wgt_ref[...], preferred_element_type=jnp.float32)
    logits_ref[...] = logits
    m = jnp.max(logits, axis=1, keepdims=True)
    p = jnp.exp(logits - m)
    p = p / jnp.sum(p, axis=1, keepdims=True)
    idx = lax.broadcasted_iota(jnp.int32, (N, E), 1)
    v0 = jnp.max(p, axis=1, keepdims=True)
    i0 = jnp.min(jnp.where(p == v0, idx, E), axis=1, keepdims=True)
    p1 = jnp.where(idx == i0, -jnp.inf, p)
    v1 = jnp.max(p1, axis=1, keepdims=True)
    i1 = jnp.min(jnp.where(p1 == v1, idx, E), axis=1, keepdims=True)
    s = v0 + v1
    w0 = v0 / s
    w1 = v1 / s
    oh0 = (idx == i0).astype(jnp.float32)
    oh1 = (idx == i1).astype(jnp.float32)
    mask = oh0 + oh1

    r = lax.broadcasted_iota(jnp.int32, (CS, CS), 0)
    c = lax.broadcasted_iota(jnp.int32, (CS, CS), 1)
    tri = (c < r).astype(jnp.float32)
    run = jnp.zeros((1, E), jnp.float32)
    chunks = []
    for ci in range(N // CS):
        mc = mask[ci * CS:(ci + 1) * CS, :]
        chunks.append(jnp.dot(tri, mc, preferred_element_type=jnp.float32)
                      + run)
        run = run + jnp.sum(mc, axis=0, keepdims=True)
    cum = jnp.concatenate(chunks, axis=0)

    pcnt = jnp.ceil(run / BM) * BM
    r8 = lax.broadcasted_iota(jnp.int32, (E, E), 0)
    c8 = lax.broadcasted_iota(jnp.int32, (E, E), 1)
    m8 = (r8 < c8).astype(jnp.float32)
    poff = jnp.dot(pcnt, m8, preferred_element_type=jnp.float32)
    poff_ref[...] = poff.astype(jnp.int32)

    pos = cum + poff
    d0 = jnp.sum(oh0 * pos, axis=1, keepdims=True)
    d1 = jnp.sum(oh1 * pos, axis=1, keepdims=True)
    lane = lax.broadcasted_iota(jnp.int32, (N, E), 1)
    dest_ref[...] = (jnp.where(lane == 0, d0, 0.0)
                     + jnp.where(lane == 1, d1, 0.0)).astype(jnp.int32)
    wpair = jnp.stack([jnp.broadcast_to(w0, (N, _WL)),
                       jnp.broadcast_to(w1, (N, _WL))], axis=1)
    wbcf_ref[...] = wpair.reshape(NA, _WL)

    be = lax.broadcasted_iota(jnp.int32, (_WL, E), 0) * BM
    le = lax.broadcasted_iota(jnp.int32, (_WL, E), 1)
    hit = (be.astype(jnp.float32) >= poff) & (le >= 1)
    eid_ref[...] = jnp.sum(hit.astype(jnp.int32), axis=1, keepdims=True)


def _router(x, Wg):
    return pl.pallas_call(
        _router_body,
        out_shape=[
            jax.ShapeDtypeStruct((N, E), jnp.float32),
            jax.ShapeDtypeStruct((NA, _WL), jnp.float32),
            jax.ShapeDtypeStruct((N, E), jnp.int32),
            jax.ShapeDtypeStruct((1, E), jnp.int32),
            jax.ShapeDtypeStruct((_WL, 1), jnp.int32),
        ],
    )(x, Wg.T)


_NC, _NS, _L = 2, 16, 16
_WL = 128
_NW = _NC * _NS
_APW = NA // _NW
_CH = 32


def _dispatch_body(x_hbm, destf_hbm, wbcf_hbm, xg_hbm, wbuf_hbm,
                   dest_v0, dest_v1, tok_v0, tok_v1, rows_v0, rows_v1,
                   wrows_v0, wrows_v1, sem_g0, sem_g1, sem_s0, sem_s1,
                   sem_w0, sem_w1):
    dest_v = (dest_v0, dest_v1)
    tok_v = (tok_v0, tok_v1)
    rows_v = (rows_v0, rows_v1)
    wrows_v = (wrows_v0, wrows_v1)
    sem_g = (sem_g0, sem_g1)
    sem_s = (sem_s0, sem_s1)
    sem_w = (sem_w0, sem_w1)
    wid = lax.axis_index("s") * _NC + lax.axis_index("c")
    base0 = wid * _APW
    nch = _APW // _CH
    gcp = [None] * nch
    scp = [None] * nch
    wcp = [None] * nch

    def fire(ch):
        b = ch % 2
        base = base0 + ch * _CH
        pltpu.sync_copy(destf_hbm.at[pl.ds(base, _CH)], dest_v[b])
        pltpu.sync_copy(wbcf_hbm.at[pl.ds(base, _CH)], wrows_v[b])
        for k in range(_CH // _L):
            tok_v[b][pl.ds(k * _L, _L)] = lax.shift_right_logical(
                base + k * _L + lax.iota(jnp.int32, _L), 1)
        gcp[ch] = pltpu.async_copy(x_hbm.at[tok_v[b]], rows_v[b], sem_g[b])

    fire(0)
    for ch in range(nch):
        if ch + 1 < nch:
            if ch - 1 >= 0:
                scp[ch - 1].wait()
                wcp[ch - 1].wait()
            fire(ch + 1)
        b = ch % 2
        gcp[ch].wait()
        scp[ch] = pltpu.async_copy(rows_v[b], xg_hbm.at[dest_v[b]], sem_s[b])
        wcp[ch] = pltpu.async_copy(wrows_v[b], wbuf_hbm.at[dest_v[b]],
                                   sem_w[b])
    if nch >= 2:
        scp[nch - 2].wait()
        wcp[nch - 2].wait()
    scp[nch - 1].wait()
    wcp[nch - 1].wait()


def _dispatch(x, destf, wbcf):
    mesh = plsc.VectorSubcoreMesh(core_axis_name="c", subcore_axis_name="s",
                                  num_cores=_NC, num_subcores=_NS)
    f = functools.partial(
        pl.kernel,
        out_type=[jax.ShapeDtypeStruct((P, D), jnp.float32),
                  jax.ShapeDtypeStruct((P, _WL), jnp.float32)],
        mesh=mesh,
        scratch_types=[
            pltpu.VMEM((_CH,), jnp.int32),
            pltpu.VMEM((_CH,), jnp.int32),
            pltpu.VMEM((_CH,), jnp.int32),
            pltpu.VMEM((_CH,), jnp.int32),
            pltpu.VMEM((_CH, D), jnp.float32),
            pltpu.VMEM((_CH, D), jnp.float32),
            pltpu.VMEM((_CH, _WL), jnp.float32),
            pltpu.VMEM((_CH, _WL), jnp.float32),
            pltpu.SemaphoreType.DMA,
            pltpu.SemaphoreType.DMA,
            pltpu.SemaphoreType.DMA,
            pltpu.SemaphoreType.DMA,
            pltpu.SemaphoreType.DMA,
            pltpu.SemaphoreType.DMA,
        ],
    )(_dispatch_body)
    return f(x, destf, wbcf)


def _ffn_body(eid_ref, xg_ref, w_ref, w1_ref, b1_ref, gamma_ref, beta_ref,
              rmean_ref, rvar_ref, w2_ref, b2_ref, obuf_ref):
    xb = xg_ref[...]
    h = lax.dot_general(xb, w1_ref[0], (((1,), (1,)), ((), ())),
                        preferred_element_type=jnp.float32)
    scale = gamma_ref[0] * lax.rsqrt(rvar_ref[0] + EPS)
    h = (h + b1_ref[0] - rmean_ref[0]) * scale + beta_ref[0]
    h = jnp.maximum(h, 0.0)
    o = lax.dot_general(h, w2_ref[0], (((1,), (1,)), ((), ())),
                        preferred_element_type=jnp.float32) + b2_ref[0]
    obuf_ref[...] = o * w_ref[:, 0:1]


def _ffn(eid, xg, wbuf, W1, b1, gamma, beta, rmean, rvar, W2, b2):
    grid_spec = pltpu.PrefetchScalarGridSpec(
        num_scalar_prefetch=1,
        grid=(NBLK,),
        in_specs=[
            pl.BlockSpec((BM, D), lambda b, eid: (b, 0)),
            pl.BlockSpec((BM, _WL), lambda b, eid: (b, 0)),
            pl.BlockSpec((1, H, D), lambda b, eid: (eid[b], 0, 0)),
            pl.BlockSpec((1, 1, H), lambda b, eid: (eid[b], 0, 0)),
            pl.BlockSpec((1, 1, H), lambda b, eid: (eid[b], 0, 0)),
            pl.BlockSpec((1, 1, H), lambda b, eid: (eid[b], 0, 0)),
            pl.BlockSpec((1, 1, H), lambda b, eid: (eid[b], 0, 0)),
            pl.BlockSpec((1, 1, H), lambda b, eid: (eid[b], 0, 0)),
            pl.BlockSpec((1, D, H), lambda b, eid: (eid[b], 0, 0)),
            pl.BlockSpec((1, 1, D), lambda b, eid: (eid[b], 0, 0)),
        ],
        out_specs=pl.BlockSpec((BM, D), lambda b, eid: (b, 0)),
    )
    return pl.pallas_call(
        _ffn_body,
        grid_spec=grid_spec,
        out_shape=jax.ShapeDtypeStruct((P, D), jnp.float32),
    )(eid, xg, wbuf, W1, b1[:, None], gamma[:, None], beta[:, None],
      rmean[:, None], rvar[:, None], W2, b2[:, None])


_TPW = N // _NW
_CT = 16


def _combine_body(x_hbm, obuf_hbm, d0_hbm, d1_hbm, fin_hbm,
                  d0_v0, d0_v1, d1_v0, d1_v1, x_v0, x_v1, r0_v0, r0_v1,
                  r1_v0, r1_v1, sem_x0, sem_x1, sem_00, sem_01, sem_10,
                  sem_11, sem_f0, sem_f1):
    d0_v = (d0_v0, d0_v1)
    d1_v = (d1_v0, d1_v1)
    x_v = (x_v0, x_v1)
    r0_v = (r0_v0, r0_v1)
    r1_v = (r1_v0, r1_v1)
    sem_x = (sem_x0, sem_x1)
    sem_0 = (sem_00, sem_01)
    sem_1 = (sem_10, sem_11)
    sem_f = (sem_f0, sem_f1)
    wid = lax.axis_index("s") * _NC + lax.axis_index("c")
    tok0 = wid * _TPW
    nch = _TPW // _CT
    gx = [None] * nch
    g0 = [None] * nch
    g1 = [None] * nch
    wb = [None] * nch

    def fire(ch):
        b = ch % 2
        base = tok0 + ch * _CT
        pltpu.sync_copy(d0_hbm.at[pl.ds(base, _CT)], d0_v[b])
        pltpu.sync_copy(d1_hbm.at[pl.ds(base, _CT)], d1_v[b])
        gx[ch] = pltpu.async_copy(x_hbm.at[pl.ds(base, _CT)], x_v[b],
                                  sem_x[b])
        g0[ch] = pltpu.async_copy(obuf_hbm.at[d0_v[b]], r0_v[b], sem_0[b])
        g1[ch] = pltpu.async_copy(obuf_hbm.at[d1_v[b]], r1_v[b], sem_1[b])

    fire(0)
    for ch in range(nch):
        if ch + 1 < nch:
            if ch - 1 >= 0:
                wb[ch - 1].wait()
            fire(ch + 1)
        b = ch % 2
        gx[ch].wait()
        g0[ch].wait()
        g1[ch].wait()

        def body(j, _):
            for k in range(D // _L):
                sl = pl.ds(k * _L, _L)
                x_v[b][j, sl] = x_v[b][j, sl] + r0_v[b][j, sl] + r1_v[b][j, sl]
            return 0

        lax.fori_loop(0, _CT, body, 0)
        wb[ch] = pltpu.async_copy(x_v[b], fin_hbm.at[pl.ds(tok0 + ch * _CT,
                                                           _CT)], sem_f[b])
    if nch >= 2:
        wb[nch - 2].wait()
    wb[nch - 1].wait()


def _combine(x, obuf, d0, d1):
    mesh = plsc.VectorSubcoreMesh(core_axis_name="c", subcore_axis_name="s",
                                  num_cores=_NC, num_subcores=_NS)
    f = functools.partial(
        pl.kernel,
        out_type=jax.ShapeDtypeStruct((N, D), jnp.float32),
        mesh=mesh,
        scratch_types=[
            pltpu.VMEM((_CT,), jnp.int32),
            pltpu.VMEM((_CT,), jnp.int32),
            pltpu.VMEM((_CT,), jnp.int32),
            pltpu.VMEM((_CT,), jnp.int32),
            pltpu.VMEM((_CT, D), jnp.float32),
            pltpu.VMEM((_CT, D), jnp.float32),
            pltpu.VMEM((_CT, D), jnp.float32),
            pltpu.VMEM((_CT, D), jnp.float32),
            pltpu.VMEM((_CT, D), jnp.float32),
            pltpu.VMEM((_CT, D), jnp.float32),
            pltpu.SemaphoreType.DMA,
            pltpu.SemaphoreType.DMA,
            pltpu.SemaphoreType.DMA,
            pltpu.SemaphoreType.DMA,
            pltpu.SemaphoreType.DMA,
            pltpu.SemaphoreType.DMA,
            pltpu.SemaphoreType.DMA,
            pltpu.SemaphoreType.DMA,
        ],
    )(_combine_body)
    return f(x, obuf, d0, d1)


def kernel(hidden_states, Wg, W1, b1, gamma, beta, rmean, rvar, W2, b2):
    x = hidden_states
    logits, wbcf, dest, poff, eidc = _router(x, Wg)

    destf = dest[:, :TOPK].reshape(-1)
    d0 = dest[:, 0]
    d1 = dest[:, 1]
    eid = eidc[:NBLK, 0]

    xg, wbuf = _dispatch(x, destf, wbcf)
    obuf = _ffn(eid, xg, wbuf, W1, b1, gamma, beta, rmean, rvar, W2, b2)
    final = _combine(x, obuf, d0, d1)
    return final, logits
